# Initial kernel scaffold; baseline (speedup 1.0000x reference)
import jax
import jax.numpy as jnp
from jax.experimental import pallas as pl


def kernel(x, edge_index, edge_weight, W1, b1, W2, b2):
    raise NotImplementedError("write your pallas kernel here")


# speedup vs baseline: 10.9575x; 10.9575x over previous
import functools

import jax
import jax.numpy as jnp
from jax import lax
from jax.experimental import pallas as pl
from jax.experimental.pallas import tpu as pltpu
from jax.experimental.pallas import tpu_sc as plsc

N_NODES = 10000
N_EDGES = 320000
NFEAT = 128
NHID = 128
NCLASS = 16

NPAD = 10240
NC = 2
NS = 16
NW = NC * NS
EPW = N_EDGES // NW
CHUNK = 80
NCHUNK = EPW // CHUNK

_MESH = plsc.VectorSubcoreMesh(core_axis_name="c", subcore_axis_name="s")


def _deg_body(dst_hbm, ew_hbm, z_hbm, out_hbm, dst_v, ew_v, acc_sh):
    c = lax.axis_index("c")
    s = lax.axis_index("s")
    wid = s * NC + c
    pltpu.sync_copy(z_hbm, acc_sh.at[pl.ds(s * 640, 640)])
    plsc.subcore_barrier()

    def chunk(i, _):
        base = pl.multiple_of(wid * EPW + i * CHUNK, 8)
        pltpu.sync_copy(dst_hbm.at[pl.ds(base, CHUNK)], dst_v)
        pltpu.sync_copy(ew_hbm.at[pl.ds(base, CHUNK)], ew_v)
        pltpu.sync_copy(ew_v, acc_sh.at[dst_v], add=True)
        return _

    lax.fori_loop(0, NCHUNK, chunk, 0)
    plsc.subcore_barrier()
    off = pl.multiple_of(s * 640, 8)
    pltpu.sync_copy(acc_sh.at[pl.ds(off, 640)], out_hbm.at[c, pl.ds(off, 640)])


_deg_kernel = functools.partial(
    pl.kernel,
    out_type=jax.ShapeDtypeStruct((NC, 10240), jnp.float32),
    mesh=_MESH,
    scratch_types=[
        pltpu.VMEM((CHUNK,), jnp.int32),
        pltpu.VMEM((CHUNK,), jnp.float32),
        pltpu.VMEM_SHARED((10240,), jnp.float32),
    ],
)(_deg_body)


def _agg_body(ndim, src_hbm, dst_hbm, ew_hbm, h_hbm, z_hbm, out_hbm,
              src_v, dst_v, ew_v, rows_v, acc_sh, sem):
    c = lax.axis_index("c")
    s = lax.axis_index("s")
    wid = s * NC + c
    rpw = NPAD // NS

    if True:
        pltpu.sync_copy(z_hbm, acc_sh.at[pl.ds(s * rpw, rpw)])
        plsc.subcore_barrier()

        def chunk(i, _):
            base = pl.multiple_of(wid * EPW + i * CHUNK, 8)
            pltpu.sync_copy(src_hbm.at[pl.ds(base, CHUNK)], src_v)
            pltpu.sync_copy(dst_hbm.at[pl.ds(base, CHUNK)], dst_v)
            pltpu.sync_copy(ew_hbm.at[pl.ds(base, CHUNK)], ew_v)
            pltpu.async_copy(h_hbm.at[src_v], rows_v, sem).wait()

            def scale(g, _):
                ew16 = ew_v[pl.ds(g * 16, 16)]
                for t in range(16):
                    e = g * 16 + t
                    w = ew16[t]
                    for j in range(ndim // 16):
                        sl = pl.ds(j * 16, 16)
                        rows_v[e, sl] = rows_v[e, sl] * w
                return _

            lax.fori_loop(0, CHUNK // 16, scale, 0)
            pltpu.sync_copy(rows_v, acc_sh.at[dst_v], add=True)
            return _

        lax.fori_loop(0, NCHUNK, chunk, 0)
        plsc.subcore_barrier()
        off = pl.multiple_of(s * rpw, 8)
        pltpu.sync_copy(acc_sh.at[pl.ds(off, rpw)], out_hbm.at[c, pl.ds(off, rpw)])


def _make_agg(ndim):
    return functools.partial(
        pl.kernel,
        out_type=jax.ShapeDtypeStruct((NC, NPAD, ndim), jnp.float32),
        mesh=_MESH,
        scratch_types=[
            pltpu.VMEM((CHUNK,), jnp.int32),
            pltpu.VMEM((CHUNK,), jnp.int32),
            pltpu.VMEM((CHUNK,), jnp.float32),
            pltpu.VMEM((CHUNK, ndim), jnp.float32),
            pltpu.VMEM_SHARED((NPAD, ndim), jnp.float32),
            pltpu.SemaphoreType.DMA,
        ],
        compiler_params=pltpu.CompilerParams(use_tc_tiling_on_sc=(ndim == NHID)),
    )(functools.partial(_agg_body, ndim))


_agg128 = _make_agg(NHID)
_agg16 = _make_agg(NCLASS)

_RB = 1000


def _tc1_body(degp_ref, x_ref, w1_ref, dis_ref, h1p_ref):
    deg = degp_ref[:, 0:1] + degp_ref[:, 1:2] + 1.0
    dis = lax.rsqrt(deg)
    h = jnp.dot(x_ref[...], w1_ref[...], preferred_element_type=jnp.float32)
    dis_ref[...] = dis
    h1p_ref[...] = h * dis


def _tc1(degp_t, x, W1):
    return pl.pallas_call(
        _tc1_body,
        grid=(N_NODES // _RB,),
        in_specs=[
            pl.BlockSpec((_RB, 2), lambda b: (b, 0)),
            pl.BlockSpec((_RB, NFEAT), lambda b: (b, 0)),
            pl.BlockSpec((NFEAT, NHID), lambda b: (0, 0)),
        ],
        out_specs=[
            pl.BlockSpec((_RB, 1), lambda b: (b, 0)),
            pl.BlockSpec((_RB, NHID), lambda b: (b, 0)),
        ],
        out_shape=[
            jax.ShapeDtypeStruct((N_NODES, 1), jnp.float32),
            jax.ShapeDtypeStruct((N_NODES, NHID), jnp.float32),
        ],
    )(degp_t, x, W1)


def _tc2_body(a0_ref, a1_ref, h1p_ref, dis_ref, b1_ref, w2_ref, h2p_ref):
    dis = dis_ref[...]
    o = dis * (a0_ref[0] + a1_ref[0] + h1p_ref[...]) + b1_ref[...]
    r = jnp.maximum(o, 0.0)
    h2 = jnp.dot(r, w2_ref[...], preferred_element_type=jnp.float32)
    h2p_ref[...] = h2 * dis


def _tc2(acc1, h1p, dis, b1, W2):
    return pl.pallas_call(
        _tc2_body,
        grid=(N_NODES // _RB,),
        in_specs=[
            pl.BlockSpec((1, _RB, NHID), lambda b: (0, b, 0)),
            pl.BlockSpec((1, _RB, NHID), lambda b: (1, b, 0)),
            pl.BlockSpec((_RB, NHID), lambda b: (b, 0)),
            pl.BlockSpec((_RB, 1), lambda b: (b, 0)),
            pl.BlockSpec((1, NHID), lambda b: (0, 0)),
            pl.BlockSpec((NHID, NCLASS), lambda b: (0, 0)),
        ],
        out_specs=pl.BlockSpec((_RB, NCLASS), lambda b: (b, 0)),
        out_shape=jax.ShapeDtypeStruct((N_NODES, NCLASS), jnp.float32),
    )(acc1, acc1, h1p, dis, b1, W2)


def _tc3_body(a0_ref, a1_ref, h2p_ref, dis_ref, b2_ref, out_ref):
    o = dis_ref[...] * (a0_ref[0] + a1_ref[0] + h2p_ref[...]) + b2_ref[...]
    m = jnp.max(o, axis=1, keepdims=True)
    z = o - m
    lse = jnp.log(jnp.sum(jnp.exp(z), axis=1, keepdims=True))
    out_ref[...] = z - lse


def _tc3(acc2, h2p, dis, b2):
    return pl.pallas_call(
        _tc3_body,
        grid=(N_NODES // _RB,),
        in_specs=[
            pl.BlockSpec((1, _RB, NCLASS), lambda b: (0, b, 0)),
            pl.BlockSpec((1, _RB, NCLASS), lambda b: (1, b, 0)),
            pl.BlockSpec((_RB, NCLASS), lambda b: (b, 0)),
            pl.BlockSpec((_RB, 1), lambda b: (b, 0)),
            pl.BlockSpec((1, NCLASS), lambda b: (0, 0)),
        ],
        out_specs=pl.BlockSpec((_RB, NCLASS), lambda b: (b, 0)),
        out_shape=jax.ShapeDtypeStruct((N_NODES, NCLASS), jnp.float32),
    )(acc2, acc2, h2p, dis, b2)


@jax.jit
def kernel(x, edge_index, edge_weight, W1, b1, W2, b2):
    src = edge_index[0].astype(jnp.int32)
    dst = edge_index[1].astype(jnp.int32)
    ew = edge_weight.astype(jnp.float32)

    z640 = jnp.zeros((640,), jnp.float32)
    z128 = jnp.zeros((NPAD // NS, NHID), jnp.float32)
    z16 = jnp.zeros((NPAD // NS, NCLASS), jnp.float32)

    degp = _deg_kernel(dst, ew, z640)
    dis, h1p = _tc1(degp[:, :N_NODES].T, x, W1)
    acc1 = _agg128(src, dst, ew, h1p, z128)
    h2p = _tc2(acc1, h1p, dis, b1.reshape(1, NHID), W2)
    acc2 = _agg16(src, dst, ew, h2p, z16)
    return _tc3(acc2, h2p, dis, b2.reshape(1, NCLASS))


# speedup vs baseline: 15.0026x; 1.3692x over previous
import functools

import jax
import jax.numpy as jnp
from jax import lax
from jax.experimental import pallas as pl
from jax.experimental.pallas import tpu as pltpu
from jax.experimental.pallas import tpu_sc as plsc

N_NODES = 10000
N_EDGES = 320000
NFEAT = 128
NHID = 128
NCLASS = 16

NPAD = 10240
NC = 2
NS = 16
NW = NC * NS
EPW = N_EDGES // NW
CHUNK = 80
NCHUNK = EPW // CHUNK
NSLOT = 4
NGROUP = (NCHUNK - 1) // NSLOT

_MESH = plsc.VectorSubcoreMesh(core_axis_name="c", subcore_axis_name="s")


DEG_NSLOT = 5


def _deg_body(dst_hbm, ew_hbm, z_hbm, out_hbm, *refs):
    NSLOT = DEG_NSLOT
    NGROUP = NCHUNK // NSLOT
    dsts = refs[0:NSLOT]
    ews = refs[NSLOT:2 * NSLOT]
    acc_sh = refs[2 * NSLOT]
    ssem = refs[2 * NSLOT + 1:3 * NSLOT + 1]
    c = lax.axis_index("c")
    s = lax.axis_index("s")
    wid = s * NC + c
    pltpu.sync_copy(z_hbm, acc_sh.at[pl.ds(s * 640, 640)])
    plsc.subcore_barrier()

    def group(g, carry):
        @pl.when(g > 0)
        def _drain():
            for b in range(NSLOT):
                pltpu.make_async_copy(ews[b], acc_sh.at[dsts[b]], ssem[b]).wait()

        for b in range(NSLOT):
            base = pl.multiple_of((wid * NCHUNK + g * NSLOT + b) * CHUNK, 8)
            pltpu.sync_copy(dst_hbm.at[pl.ds(base, CHUNK)], dsts[b])
            pltpu.sync_copy(ew_hbm.at[pl.ds(base, CHUNK)], ews[b])
            pltpu.async_copy(ews[b], acc_sh.at[dsts[b]], ssem[b], add=True)
        return carry

    lax.fori_loop(0, NGROUP, group, 0)
    for b in range(NSLOT):
        pltpu.make_async_copy(ews[b], acc_sh.at[dsts[b]], ssem[b]).wait()
    plsc.subcore_barrier()
    off = pl.multiple_of(s * 640, 8)
    pltpu.sync_copy(acc_sh.at[pl.ds(off, 640)], out_hbm.at[c, pl.ds(off, 640)])


_deg_kernel = functools.partial(
    pl.kernel,
    out_type=jax.ShapeDtypeStruct((NC, 10240), jnp.float32),
    mesh=_MESH,
    scratch_types=(
        [pltpu.VMEM((CHUNK,), jnp.int32) for _ in range(DEG_NSLOT)]
        + [pltpu.VMEM((CHUNK,), jnp.float32) for _ in range(DEG_NSLOT)]
        + [pltpu.VMEM_SHARED((10240,), jnp.float32)]
        + [pltpu.SemaphoreType.DMA for _ in range(DEG_NSLOT)]
    ),
)(_deg_body)


def _agg_body(ndim, nslot, src_hbm, dst_hbm, ew_hbm, h_hbm, z_hbm, out_hbm, *refs):
    ngroup = NCHUNK // nslot
    rem = NCHUNK % nslot
    srcs = refs[0:nslot]
    dsts = refs[nslot:2 * nslot]
    ews = refs[2 * nslot:3 * nslot]
    rows = refs[3 * nslot:4 * nslot]
    acc_sh = refs[4 * nslot]
    gsem = refs[4 * nslot + 1:5 * nslot + 1]
    ssem = refs[5 * nslot + 1:6 * nslot + 1]
    c = lax.axis_index("c")
    s = lax.axis_index("s")
    wid = s * NC + c
    rpw = NPAD // NS

    pltpu.sync_copy(z_hbm, acc_sh.at[pl.ds(s * rpw, rpw)])
    plsc.subcore_barrier()

    def group(g, carry):
        @pl.when(g > 0)
        def _drain():
            for b in range(nslot):
                pltpu.make_async_copy(rows[b], acc_sh.at[dsts[b]], ssem[b]).wait()

        for b in range(nslot):
            base = pl.multiple_of((wid * NCHUNK + g * nslot + b) * CHUNK, 8)
            pltpu.sync_copy(src_hbm.at[pl.ds(base, CHUNK)], srcs[b])
            pltpu.sync_copy(dst_hbm.at[pl.ds(base, CHUNK)], dsts[b])
            pltpu.sync_copy(ew_hbm.at[pl.ds(base, CHUNK)], ews[b])
            pltpu.async_copy(h_hbm.at[srcs[b]], rows[b], gsem[b])
        for b in range(nslot):
            pltpu.make_async_copy(h_hbm.at[srcs[b]], rows[b], gsem[b]).wait()

            def scale(q, _, b=b):
                ew16 = ews[b][pl.ds(q * 16, 16)]
                for t in range(16):
                    e = q * 16 + t
                    w = ew16[t]
                    for j in range(ndim // 16):
                        sl = pl.ds(j * 16, 16)
                        rows[b][e, sl] = rows[b][e, sl] * w
                return _

            lax.fori_loop(0, CHUNK // 16, scale, 0)
            pltpu.async_copy(rows[b], acc_sh.at[dsts[b]], ssem[b], add=True)
        return carry

    lax.fori_loop(0, ngroup, group, 0)
    for b in range(nslot):
        pltpu.make_async_copy(rows[b], acc_sh.at[dsts[b]], ssem[b]).wait()

    for b in range(rem):
        base = pl.multiple_of((wid * NCHUNK + ngroup * nslot + b) * CHUNK, 8)
        pltpu.sync_copy(src_hbm.at[pl.ds(base, CHUNK)], srcs[b])
        pltpu.sync_copy(dst_hbm.at[pl.ds(base, CHUNK)], dsts[b])
        pltpu.sync_copy(ew_hbm.at[pl.ds(base, CHUNK)], ews[b])
        pltpu.async_copy(h_hbm.at[srcs[b]], rows[b], gsem[b]).wait()

        def scale_tail(q, _, b=b):
            ew16 = ews[b][pl.ds(q * 16, 16)]
            for t in range(16):
                e = q * 16 + t
                w = ew16[t]
                for j in range(ndim // 16):
                    sl = pl.ds(j * 16, 16)
                    rows[b][e, sl] = rows[b][e, sl] * w
            return _

        lax.fori_loop(0, CHUNK // 16, scale_tail, 0)
        pltpu.sync_copy(rows[b], acc_sh.at[dsts[b]], add=True)
    plsc.subcore_barrier()
    off = pl.multiple_of(s * rpw, 8)
    pltpu.sync_copy(acc_sh.at[pl.ds(off, rpw)], out_hbm.at[c, pl.ds(off, rpw)])


def _make_agg(ndim, nslot):
    return functools.partial(
        pl.kernel,
        out_type=jax.ShapeDtypeStruct((NC, NPAD, ndim), jnp.float32),
        mesh=_MESH,
        scratch_types=(
            [pltpu.VMEM((CHUNK,), jnp.int32) for _ in range(2 * nslot)]
            + [pltpu.VMEM((CHUNK,), jnp.float32) for _ in range(nslot)]
            + [pltpu.VMEM((CHUNK, ndim), jnp.float32) for _ in range(nslot)]
            + [pltpu.VMEM_SHARED((NPAD, ndim), jnp.float32)]
            + [pltpu.SemaphoreType.DMA for _ in range(2 * nslot)]
        ),
        compiler_params=pltpu.CompilerParams(use_tc_tiling_on_sc=(ndim == NHID)),
    )(functools.partial(_agg_body, ndim, nslot))


_agg128 = _make_agg(NHID, 4)
_agg16 = _make_agg(NCLASS, 5)

_RB = 1000


def _tc0_body(x_ref, w1_ref, h1_ref):
    h1_ref[...] = jnp.dot(x_ref[...], w1_ref[...],
                          preferred_element_type=jnp.float32)


def _tc0(x, W1):
    return pl.pallas_call(
        _tc0_body,
        grid=(N_NODES // _RB,),
        in_specs=[
            pl.BlockSpec((_RB, NFEAT), lambda b: (b, 0)),
            pl.BlockSpec((NFEAT, NHID), lambda b: (0, 0)),
        ],
        out_specs=pl.BlockSpec((_RB, NHID), lambda b: (b, 0)),
        out_shape=jax.ShapeDtypeStruct((N_NODES, NHID), jnp.float32),
    )(x, W1)


def _tc1_body(degp_ref, h1_ref, dis_ref, h1p_ref):
    deg = degp_ref[:, 0:1] + degp_ref[:, 1:2] + 1.0
    dis = lax.rsqrt(deg)
    dis_ref[...] = dis
    h1p_ref[...] = h1_ref[...] * dis


def _tc1(degp_t, h1):
    return pl.pallas_call(
        _tc1_body,
        grid=(N_NODES // _RB,),
        in_specs=[
            pl.BlockSpec((_RB, 2), lambda b: (b, 0)),
            pl.BlockSpec((_RB, NHID), lambda b: (b, 0)),
        ],
        out_specs=[
            pl.BlockSpec((_RB, 1), lambda b: (b, 0)),
            pl.BlockSpec((_RB, NHID), lambda b: (b, 0)),
        ],
        out_shape=[
            jax.ShapeDtypeStruct((N_NODES, 1), jnp.float32),
            jax.ShapeDtypeStruct((N_NODES, NHID), jnp.float32),
        ],
    )(degp_t, h1)


def _tc2_body(a0_ref, a1_ref, h1p_ref, dis_ref, b1_ref, w2_ref, h2p_ref):
    dis = dis_ref[...]
    o = dis * (a0_ref[0] + a1_ref[0] + h1p_ref[...]) + b1_ref[...]
    r = jnp.maximum(o, 0.0)
    h2 = jnp.dot(r, w2_ref[...], preferred_element_type=jnp.float32)
    h2p_ref[...] = h2 * dis


def _tc2(acc1, h1p, dis, b1, W2):
    return pl.pallas_call(
        _tc2_body,
        grid=(N_NODES // _RB,),
        in_specs=[
            pl.BlockSpec((1, _RB, NHID), lambda b: (0, b, 0)),
            pl.BlockSpec((1, _RB, NHID), lambda b: (1, b, 0)),
            pl.BlockSpec((_RB, NHID), lambda b: (b, 0)),
            pl.BlockSpec((_RB, 1), lambda b: (b, 0)),
            pl.BlockSpec((1, NHID), lambda b: (0, 0)),
            pl.BlockSpec((NHID, NCLASS), lambda b: (0, 0)),
        ],
        out_specs=pl.BlockSpec((_RB, NCLASS), lambda b: (b, 0)),
        out_shape=jax.ShapeDtypeStruct((N_NODES, NCLASS), jnp.float32),
    )(acc1, acc1, h1p, dis, b1, W2)


def _tc3_body(a0_ref, a1_ref, h2p_ref, dis_ref, b2_ref, out_ref):
    o = dis_ref[...] * (a0_ref[0] + a1_ref[0] + h2p_ref[...]) + b2_ref[...]
    m = jnp.max(o, axis=1, keepdims=True)
    z = o - m
    lse = jnp.log(jnp.sum(jnp.exp(z), axis=1, keepdims=True))
    out_ref[...] = z - lse


def _tc3(acc2, h2p, dis, b2):
    return pl.pallas_call(
        _tc3_body,
        grid=(N_NODES // _RB,),
        in_specs=[
            pl.BlockSpec((1, _RB, NCLASS), lambda b: (0, b, 0)),
            pl.BlockSpec((1, _RB, NCLASS), lambda b: (1, b, 0)),
            pl.BlockSpec((_RB, NCLASS), lambda b: (b, 0)),
            pl.BlockSpec((_RB, 1), lambda b: (b, 0)),
            pl.BlockSpec((1, NCLASS), lambda b: (0, 0)),
        ],
        out_specs=pl.BlockSpec((_RB, NCLASS), lambda b: (b, 0)),
        out_shape=jax.ShapeDtypeStruct((N_NODES, NCLASS), jnp.float32),
    )(acc2, acc2, h2p, dis, b2)


@jax.jit
def kernel(x, edge_index, edge_weight, W1, b1, W2, b2):
    src = edge_index[0].astype(jnp.int32)
    dst = edge_index[1].astype(jnp.int32)
    ew = edge_weight.astype(jnp.float32)

    z640 = jnp.zeros((640,), jnp.float32)
    z128 = jnp.zeros((NPAD // NS, NHID), jnp.float32)
    z16 = jnp.zeros((NPAD // NS, NCLASS), jnp.float32)

    h1 = _tc0(x, W1)
    degp = _deg_kernel(dst, ew, z640)
    dis, h1p = _tc1(degp[:, :N_NODES].T, h1)
    acc1 = _agg128(src, dst, ew, h1p, z128)
    h2p = _tc2(acc1, h1p, dis, b1.reshape(1, NHID), W2)
    acc2 = _agg16(src, dst, ew, h2p, z16)
    return _tc3(acc2, h2p, dis, b2.reshape(1, NCLASS))


# speedup vs baseline: 30.6783x; 2.0449x over previous
import functools

import jax
import jax.numpy as jnp
from jax import lax
from jax.experimental import pallas as pl
from jax.experimental.pallas import tpu as pltpu
from jax.experimental.pallas import tpu_sc as plsc

N_NODES = 10000
N_EDGES = 320000
NFEAT = 128
NHID = 128
NCLASS = 16

NPAD = 10240
NC = 2
NS = 16
NW = NC * NS
EPW = N_EDGES // NW
CHUNK = 80
NCHUNK = EPW // CHUNK
NSLOT = 4
NGROUP = (NCHUNK - 1) // NSLOT

_MESH = plsc.VectorSubcoreMesh(core_axis_name="c", subcore_axis_name="s")


DEG_NSLOT = 5


def _deg_body(dst_hbm, ew_hbm, z_hbm, out_hbm, *refs):
    NSLOT = DEG_NSLOT
    NGROUP = NCHUNK // NSLOT
    dsts = refs[0:NSLOT]
    ews = refs[NSLOT:2 * NSLOT]
    acc_sh = refs[2 * NSLOT]
    ssem = refs[2 * NSLOT + 1:3 * NSLOT + 1]
    isem = refs[3 * NSLOT + 1:4 * NSLOT + 1]
    c = lax.axis_index("c")
    s = lax.axis_index("s")
    wid = s * NC + c
    pltpu.sync_copy(z_hbm, acc_sh.at[pl.ds(s * 640, 640)])
    plsc.subcore_barrier()

    def group(g, carry):
        for b in range(NSLOT):
            @pl.when(g > 0)
            def _drain(b=b):
                pltpu.make_async_copy(ews[b], acc_sh.at[dsts[b]], ssem[b]).wait()

            base = pl.multiple_of((wid * NCHUNK + g * NSLOT + b) * CHUNK, 8)
            pltpu.async_copy(dst_hbm.at[pl.ds(base, CHUNK)], dsts[b], isem[b])
            pltpu.async_copy(ew_hbm.at[pl.ds(base, CHUNK)], ews[b], isem[b])
        for b in range(NSLOT):
            base = pl.multiple_of((wid * NCHUNK + g * NSLOT + b) * CHUNK, 8)
            pltpu.make_async_copy(dst_hbm.at[pl.ds(base, CHUNK)], dsts[b], isem[b]).wait()
            pltpu.make_async_copy(ew_hbm.at[pl.ds(base, CHUNK)], ews[b], isem[b]).wait()
            pltpu.async_copy(ews[b], acc_sh.at[dsts[b]], ssem[b], add=True)
        return carry

    lax.fori_loop(0, NGROUP, group, 0)
    for b in range(NSLOT):
        pltpu.make_async_copy(ews[b], acc_sh.at[dsts[b]], ssem[b]).wait()
    plsc.subcore_barrier()
    off = pl.multiple_of(s * 640, 8)
    pltpu.sync_copy(acc_sh.at[pl.ds(off, 640)], out_hbm.at[c, pl.ds(off, 640)])


_deg_kernel = functools.partial(
    pl.kernel,
    out_type=jax.ShapeDtypeStruct((NC, 10240), jnp.float32),
    mesh=_MESH,
    scratch_types=(
        [pltpu.VMEM((CHUNK,), jnp.int32) for _ in range(DEG_NSLOT)]
        + [pltpu.VMEM((CHUNK,), jnp.float32) for _ in range(DEG_NSLOT)]
        + [pltpu.VMEM_SHARED((10240,), jnp.float32)]
        + [pltpu.SemaphoreType.DMA for _ in range(2 * DEG_NSLOT)]
    ),
)(_deg_body)


def _agg_body(ndim, nslot, src_hbm, dst_hbm, ew_hbm, h_hbm, z_hbm, out_hbm, *refs):
    ngroup = NCHUNK // nslot
    rem = NCHUNK % nslot
    srcs = refs[0:nslot]
    dsts = refs[nslot:2 * nslot]
    ews = refs[2 * nslot:3 * nslot]
    rows = refs[3 * nslot:4 * nslot]
    acc_sh = refs[4 * nslot]
    gsem = refs[4 * nslot + 1:5 * nslot + 1]
    ssem = refs[5 * nslot + 1:6 * nslot + 1]
    isem = refs[6 * nslot + 1:7 * nslot + 1]
    c = lax.axis_index("c")
    s = lax.axis_index("s")
    wid = s * NC + c
    rpw = NPAD // NS

    pltpu.sync_copy(z_hbm, acc_sh.at[pl.ds(s * rpw, rpw)])
    plsc.subcore_barrier()

    def group(g, carry):
        for b in range(nslot):
            @pl.when(g > 0)
            def _drain(b=b):
                pltpu.make_async_copy(rows[b], acc_sh.at[dsts[b]], ssem[b]).wait()

            base = pl.multiple_of((wid * NCHUNK + g * nslot + b) * CHUNK, 8)
            pltpu.async_copy(src_hbm.at[pl.ds(base, CHUNK)], srcs[b], isem[b])
            pltpu.async_copy(dst_hbm.at[pl.ds(base, CHUNK)], dsts[b], isem[b])
            pltpu.async_copy(ew_hbm.at[pl.ds(base, CHUNK)], ews[b], isem[b])
        for b in range(nslot):
            base = pl.multiple_of((wid * NCHUNK + g * nslot + b) * CHUNK, 8)
            pltpu.make_async_copy(src_hbm.at[pl.ds(base, CHUNK)], srcs[b], isem[b]).wait()
            pltpu.make_async_copy(dst_hbm.at[pl.ds(base, CHUNK)], dsts[b], isem[b]).wait()
            pltpu.make_async_copy(ew_hbm.at[pl.ds(base, CHUNK)], ews[b], isem[b]).wait()
            pltpu.async_copy(h_hbm.at[srcs[b]], rows[b], gsem[b])
        for b in range(nslot):
            pltpu.make_async_copy(h_hbm.at[srcs[b]], rows[b], gsem[b]).wait()

            def scale(q, _, b=b):
                ew16 = ews[b][pl.ds(q * 16, 16)]
                for t in range(16):
                    e = q * 16 + t
                    w = ew16[t]
                    for j in range(ndim // 16):
                        sl = pl.ds(j * 16, 16)
                        rows[b][e, sl] = rows[b][e, sl] * w
                return _

            lax.fori_loop(0, CHUNK // 16, scale, 0)
            pltpu.async_copy(rows[b], acc_sh.at[dsts[b]], ssem[b], add=True)
        return carry

    lax.fori_loop(0, ngroup, group, 0)
    for b in range(nslot):
        pltpu.make_async_copy(rows[b], acc_sh.at[dsts[b]], ssem[b]).wait()

    for b in range(rem):
        base = pl.multiple_of((wid * NCHUNK + ngroup * nslot + b) * CHUNK, 8)
        pltpu.sync_copy(src_hbm.at[pl.ds(base, CHUNK)], srcs[b])
        pltpu.sync_copy(dst_hbm.at[pl.ds(base, CHUNK)], dsts[b])
        pltpu.sync_copy(ew_hbm.at[pl.ds(base, CHUNK)], ews[b])
        pltpu.async_copy(h_hbm.at[srcs[b]], rows[b], gsem[b]).wait()

        def scale_tail(q, _, b=b):
            ew16 = ews[b][pl.ds(q * 16, 16)]
            for t in range(16):
                e = q * 16 + t
                w = ew16[t]
                for j in range(ndim // 16):
                    sl = pl.ds(j * 16, 16)
                    rows[b][e, sl] = rows[b][e, sl] * w
            return _

        lax.fori_loop(0, CHUNK // 16, scale_tail, 0)
        pltpu.sync_copy(rows[b], acc_sh.at[dsts[b]], add=True)
    plsc.subcore_barrier()
    off = pl.multiple_of(s * rpw, 8)
    pltpu.sync_copy(acc_sh.at[pl.ds(off, rpw)], out_hbm.at[c, pl.ds(off, rpw)])


def _make_agg(ndim, nslot):
    return functools.partial(
        pl.kernel,
        out_type=jax.ShapeDtypeStruct((NC, NPAD, ndim), jnp.float32),
        mesh=_MESH,
        scratch_types=(
            [pltpu.VMEM((CHUNK,), jnp.int32) for _ in range(2 * nslot)]
            + [pltpu.VMEM((CHUNK,), jnp.float32) for _ in range(nslot)]
            + [pltpu.VMEM((CHUNK, ndim), jnp.float32) for _ in range(nslot)]
            + [pltpu.VMEM_SHARED((NPAD, ndim), jnp.float32)]
            + [pltpu.SemaphoreType.DMA for _ in range(3 * nslot)]
        ),
        compiler_params=pltpu.CompilerParams(use_tc_tiling_on_sc=(ndim == NHID)),
    )(functools.partial(_agg_body, ndim, nslot))


_agg128 = _make_agg(NHID, 4)
_agg16 = _make_agg(NCLASS, 5)

_RB = 1024


def _tc0_body(x_ref, w1_ref, h1_ref):
    h1_ref[...] = jnp.dot(x_ref[...], w1_ref[...],
                          preferred_element_type=jnp.float32)


def _tc0(x, W1):
    return pl.pallas_call(
        _tc0_body,
        grid=(-(-N_NODES // _RB),),
        in_specs=[
            pl.BlockSpec((_RB, NFEAT), lambda b: (b, 0)),
            pl.BlockSpec((NFEAT, NHID), lambda b: (0, 0)),
        ],
        out_specs=pl.BlockSpec((_RB, NHID), lambda b: (b, 0)),
        out_shape=jax.ShapeDtypeStruct((N_NODES, NHID), jnp.float32),
    )(x, W1)


def _tc1_body(d_ref, h1_ref, dis_ref, h1p_ref):
    deg = d_ref[0] + d_ref[1] + 1.0
    dis = lax.rsqrt(deg).reshape(_RB, 1)
    dis_ref[...] = dis
    h1p_ref[...] = h1_ref[...] * dis


def _tc1(degp, h1):
    return pl.pallas_call(
        _tc1_body,
        grid=(10240 // _RB,),
        in_specs=[
            pl.BlockSpec((2, _RB), lambda b: (0, b)),
            pl.BlockSpec((_RB, NHID), lambda b: (b, 0)),
        ],
        out_specs=[
            pl.BlockSpec((_RB, 1), lambda b: (b, 0)),
            pl.BlockSpec((_RB, NHID), lambda b: (b, 0)),
        ],
        out_shape=[
            jax.ShapeDtypeStruct((N_NODES, 1), jnp.float32),
            jax.ShapeDtypeStruct((N_NODES, NHID), jnp.float32),
        ],
    )(degp, h1)


def _tc2_body(a0_ref, a1_ref, h1p_ref, dis_ref, b1_ref, w2_ref, h2p_ref):
    dis = dis_ref[...]
    o = dis * (a0_ref[0] + a1_ref[0] + h1p_ref[...]) + b1_ref[...]
    r = jnp.maximum(o, 0.0)
    h2 = jnp.dot(r, w2_ref[...], preferred_element_type=jnp.float32)
    h2p_ref[...] = h2 * dis


def _tc2(acc1, h1p, dis, b1, W2):
    return pl.pallas_call(
        _tc2_body,
        grid=(-(-N_NODES // _RB),),
        in_specs=[
            pl.BlockSpec((1, _RB, NHID), lambda b: (0, b, 0)),
            pl.BlockSpec((1, _RB, NHID), lambda b: (1, b, 0)),
            pl.BlockSpec((_RB, NHID), lambda b: (b, 0)),
            pl.BlockSpec((_RB, 1), lambda b: (b, 0)),
            pl.BlockSpec((1, NHID), lambda b: (0, 0)),
            pl.BlockSpec((NHID, NCLASS), lambda b: (0, 0)),
        ],
        out_specs=pl.BlockSpec((_RB, NCLASS), lambda b: (b, 0)),
        out_shape=jax.ShapeDtypeStruct((N_NODES, NCLASS), jnp.float32),
    )(acc1, acc1, h1p, dis, b1, W2)


def _tc3_body(a0_ref, a1_ref, h2p_ref, dis_ref, b2_ref, out_ref):
    o = dis_ref[...] * (a0_ref[0] + a1_ref[0] + h2p_ref[...]) + b2_ref[...]
    m = jnp.max(o, axis=1, keepdims=True)
    z = o - m
    lse = jnp.log(jnp.sum(jnp.exp(z), axis=1, keepdims=True))
    out_ref[...] = z - lse


def _tc3(acc2, h2p, dis, b2):
    return pl.pallas_call(
        _tc3_body,
        grid=(-(-N_NODES // _RB),),
        in_specs=[
            pl.BlockSpec((1, _RB, NCLASS), lambda b: (0, b, 0)),
            pl.BlockSpec((1, _RB, NCLASS), lambda b: (1, b, 0)),
            pl.BlockSpec((_RB, NCLASS), lambda b: (b, 0)),
            pl.BlockSpec((_RB, 1), lambda b: (b, 0)),
            pl.BlockSpec((1, NCLASS), lambda b: (0, 0)),
        ],
        out_specs=pl.BlockSpec((_RB, NCLASS), lambda b: (b, 0)),
        out_shape=jax.ShapeDtypeStruct((N_NODES, NCLASS), jnp.float32),
    )(acc2, acc2, h2p, dis, b2)


@jax.jit
def kernel(x, edge_index, edge_weight, W1, b1, W2, b2):
    src = edge_index[0].astype(jnp.int32)
    dst = edge_index[1].astype(jnp.int32)
    ew = edge_weight.astype(jnp.float32)

    z640 = jnp.zeros((640,), jnp.float32)
    z128 = jnp.zeros((NPAD // NS, NHID), jnp.float32)
    z16 = jnp.zeros((NPAD // NS, NCLASS), jnp.float32)

    h1 = _tc0(x, W1)
    degp = _deg_kernel(dst, ew, z640)
    dis, h1p = _tc1(degp, h1)
    acc1 = _agg128(src, dst, ew, h1p, z128)
    h2p = _tc2(acc1, h1p, dis, b1.reshape(1, NHID), W2)
    acc2 = _agg16(src, dst, ew, h2p, z16)
    return _tc3(acc2, h2p, dis, b2.reshape(1, NCLASS))
